# baseline (speedup 1.0000x reference)
import functools
import jax
import jax.numpy as jnp
from jax.experimental import pallas as pl
from jax.experimental.pallas import tpu as pltpu

_N0 = 25000
_N1 = 25000
_N = _N0 + _N1
_D = 64
_HEADS = 2
_CONVS = 2


def _in_proj_body(f0_ref, f1_ref, w0_ref, b0_ref, w1_ref, b1_ref, h_ref, s_ref):
    i = pl.program_id(0)
    nb = pl.num_programs(0)
    half = nb // 2

    @pl.when(i < half)
    def _():
        h = jnp.dot(f0_ref[...], w0_ref[...], preferred_element_type=jnp.float32)
        h = h + b0_ref[...]
        h_ref[...] = h
        s_ref[...] = jnp.sum(h, axis=1, keepdims=True)

    @pl.when(i >= half)
    def _():
        h = jnp.dot(f1_ref[...], w1_ref[...], preferred_element_type=jnp.float32)
        h = h + b1_ref[...]
        h_ref[...] = h
        s_ref[...] = jnp.sum(h, axis=1, keepdims=True)


def _in_proj(feat0, feat1, W1_0, b1_0, W1_1, b1_1):
    blk = 1000
    nb0 = _N0 // blk
    nb = 2 * nb0
    grid = (nb,)

    def f0_idx(i):
        return (jnp.minimum(i, nb0 - 1), 0)

    def f1_idx(i):
        return (jnp.maximum(i - nb0, 0), 0)

    h, s = pl.pallas_call(
        _in_proj_body,
        grid=grid,
        in_specs=[
            pl.BlockSpec((blk, 128), f0_idx),
            pl.BlockSpec((blk, 128), f1_idx),
            pl.BlockSpec((128, _D), lambda i: (0, 0)),
            pl.BlockSpec((_D,), lambda i: (0,)),
            pl.BlockSpec((128, _D), lambda i: (0, 0)),
            pl.BlockSpec((_D,), lambda i: (0,)),
        ],
        out_specs=[
            pl.BlockSpec((blk, _D), lambda i: (i, 0)),
            pl.BlockSpec((blk, 1), lambda i: (i, 0)),
        ],
        out_shape=[
            jax.ShapeDtypeStruct((_N, _D), jnp.float32),
            jax.ShapeDtypeStruct((_N, 1), jnp.float32),
        ],
    )(feat0, feat1, W1_0, b1_0, W1_1, b1_1)
    return h, s[:, 0]


def _out_proj_body(z0_ref, z1_ref, w2_ref, b2_ref, w3_ref, b3_ref, enc_ref, log_ref):
    z = jnp.concatenate([z0_ref[...], z1_ref[...]], axis=1)
    z = jnp.maximum(z, 0.0)
    e = jnp.dot(z, w2_ref[...], preferred_element_type=jnp.float32) + b2_ref[...]
    enc_ref[...] = e
    e = jnp.maximum(e, 0.0)
    log_ref[...] = jnp.dot(e, w3_ref[...], preferred_element_type=jnp.float32) + b3_ref[...]


def _out_proj(z0, z1, W2, b2, W3, b3):
    blk = 1000
    nb = _N // blk
    enc, log = pl.pallas_call(
        _out_proj_body,
        grid=(nb,),
        in_specs=[
            pl.BlockSpec((blk, _D), lambda i: (i, 0)),
            pl.BlockSpec((blk, _D), lambda i: (i, 0)),
            pl.BlockSpec((2 * _D, _D), lambda i: (0, 0)),
            pl.BlockSpec((_D,), lambda i: (0,)),
            pl.BlockSpec((_D, 16), lambda i: (0, 0)),
            pl.BlockSpec((16,), lambda i: (0,)),
        ],
        out_specs=[
            pl.BlockSpec((blk, _D), lambda i: (i, 0)),
            pl.BlockSpec((blk, 16), lambda i: (i, 0)),
        ],
        out_shape=[
            jax.ShapeDtypeStruct((_N, _D), jnp.float32),
            jax.ShapeDtypeStruct((_N, 16), jnp.float32),
        ],
    )(z0, z1, W2, b2, W3, b3)
    return enc, log


def _conv_jnp(h, s, src, dst, al_c, ar_c):
    u = s[src]
    v = s[dst]
    x = al_c[0][None, :] * u[:, None] + ar_c[0][None, :] * v[:, None]
    x = jnp.where(x >= 0.0, x, 0.2 * x)
    ex = jnp.exp(x)
    esum = jax.ops.segment_sum(ex, dst, num_segments=_N)
    numer = jax.ops.segment_sum(h[src] * ex, dst, num_segments=_N)
    out = numer / (esum + 1e-9)
    return out


def kernel(feat0, feat1, edge_index, e_feat, W1_0, b1_0, W1_1, b1_1, al, ar, W2, b2, W3, b3):
    src = edge_index[0]
    dst = edge_index[1]
    h, s = _in_proj(feat0, feat1, W1_0, b1_0, W1_1, b1_1)
    zs = []
    for hd in range(_HEADS):
        x = h
        sx = s
        for c in range(_CONVS):
            x = _conv_jnp(x, sx, src, dst, al[hd, c], ar[hd, c])
            sx = jnp.sum(x, axis=1)
        zs.append(h + x)
    enc, log = _out_proj(zs[0], zs[1], W2, b2, W3, b3)
    return (log, enc)


# speedup vs baseline: 1.9304x; 1.9304x over previous
import functools
import jax
import jax.numpy as jnp
from jax import lax
from jax.experimental import pallas as pl
from jax.experimental.pallas import tpu as pltpu
from jax.experimental.pallas import tpu_sc as plsc

_N0 = 25000
_N1 = 25000
_N = _N0 + _N1
_E = 800000
_D = 64
_G = 4
_W = _D // _G
_HEADS = 2
_CONVS = 2

_NC = 2
_NS = 16
_LM = 80
_RC = 8
_C = _LM * _RC
_ER = _E // _LM
_NCHUNK = _E // _C
_KPS = -(-_NCHUNK // _NS)
_HALF = _N // _NC
_ZR = 1568
_AROWS = _ZR * _NS
_FR = 1560


def _in_proj_body(f0_ref, f1_ref, w0_ref, b0_ref, w1_ref, b1_ref, *outs):
    i = pl.program_id(0)
    nb = pl.num_programs(0)
    half = nb // 2
    h_refs = outs[:_G]
    s_ref = outs[_G]

    def emit(h):
        for g in range(_G):
            h_refs[g][...] = h[:, g * _W:(g + 1) * _W]
        s_ref[...] = jnp.sum(h, axis=1, keepdims=True)

    @pl.when(i < half)
    def _():
        emit(jnp.dot(f0_ref[...], w0_ref[...], preferred_element_type=jnp.float32)
             + b0_ref[...])

    @pl.when(i >= half)
    def _():
        emit(jnp.dot(f1_ref[...], w1_ref[...], preferred_element_type=jnp.float32)
             + b1_ref[...])


def _in_proj(feat0, feat1, W1_0, b1_0, W1_1, b1_1):
    blk = 1000
    nb0 = _N0 // blk
    nb = 2 * nb0

    def f0_idx(i):
        return (jnp.minimum(i, nb0 - 1), 0)

    def f1_idx(i):
        return (jnp.maximum(i - nb0, 0), 0)

    outs = pl.pallas_call(
        _in_proj_body,
        grid=(nb,),
        in_specs=[
            pl.BlockSpec((blk, 128), f0_idx),
            pl.BlockSpec((blk, 128), f1_idx),
            pl.BlockSpec((128, _D), lambda i: (0, 0)),
            pl.BlockSpec((_D,), lambda i: (0,)),
            pl.BlockSpec((128, _D), lambda i: (0, 0)),
            pl.BlockSpec((_D,), lambda i: (0,)),
        ],
        out_specs=[pl.BlockSpec((blk, _W), lambda i: (i, 0)) for _ in range(_G)]
        + [pl.BlockSpec((blk, 1), lambda i: (i, 0))],
        out_shape=[jax.ShapeDtypeStruct((_N, _W), jnp.float32) for _ in range(_G)]
        + [jax.ShapeDtypeStruct((_N, 1), jnp.float32)],
    )(feat0, feat1, W1_0, b1_0, W1_1, b1_1)
    return list(outs[:_G]), outs[_G]


def _exk_body(u_ref, v_ref, al_ref, ar_ref, *ex_refs):
    x = u_ref[...] * al_ref[...] + v_ref[...] * ar_ref[...]
    x = jnp.where(x >= 0.0, x, 0.2 * x)
    ex = jnp.exp(x)
    for g in range(_G):
        ex_refs[g][...] = ex[:, g * _W:(g + 1) * _W]


def _exk(ssrc, sdst, al_c, ar_c):
    blk = 2000
    nb = _E // blk
    exs = pl.pallas_call(
        _exk_body,
        grid=(nb,),
        in_specs=[
            pl.BlockSpec((blk, 1), lambda i: (i, 0)),
            pl.BlockSpec((blk, 1), lambda i: (i, 0)),
            pl.BlockSpec((1, _D), lambda i: (0, 0)),
            pl.BlockSpec((1, _D), lambda i: (0, 0)),
        ],
        out_specs=[pl.BlockSpec((blk, _W), lambda i: (i, 0)) for _ in range(_G)],
        out_shape=[jax.ShapeDtypeStruct((_E, _W), jnp.float32) for _ in range(_G)],
    )(ssrc, sdst, al_c, ar_c)
    return list(exs)


def _prep_body(*refs):
    e_refs = refs[:_G]
    n_refs = refs[_G:2 * _G]
    x_refs = refs[2 * _G:3 * _G]
    s_ref = refs[3 * _G]
    s = None
    for g in range(_G):
        x = n_refs[g][...] / (e_refs[g][...] + 1e-9)
        x_refs[g][...] = x
        r = jnp.sum(x, axis=1, keepdims=True)
        s = r if s is None else s + r
    s_ref[...] = s


def _prep(esums, numers):
    blk = 2000
    nb = _N // blk
    dspec = pl.BlockSpec((blk, _W), lambda i: (i, 0))
    outs = pl.pallas_call(
        _prep_body,
        grid=(nb,),
        in_specs=[dspec] * (2 * _G),
        out_specs=[dspec] * _G + [pl.BlockSpec((blk, 1), lambda i: (i, 0))],
        out_shape=[jax.ShapeDtypeStruct((_N, _W), jnp.float32) for _ in range(_G)]
        + [jax.ShapeDtypeStruct((_N, 1), jnp.float32)],
    )(*esums, *numers)
    return list(outs[:_G]), outs[_G]


def _out_proj_body(*refs):
    h_refs = refs[:_G]
    e0_refs = refs[_G:2 * _G]
    n0_refs = refs[2 * _G:3 * _G]
    e1_refs = refs[3 * _G:4 * _G]
    n1_refs = refs[4 * _G:5 * _G]
    w2_ref, b2_ref, w3_ref, b3_ref, enc_ref, log_ref = refs[5 * _G:]
    cols = []
    for g in range(_G):
        cols.append(h_refs[g][...] + n0_refs[g][...] / (e0_refs[g][...] + 1e-9))
    for g in range(_G):
        cols.append(h_refs[g][...] + n1_refs[g][...] / (e1_refs[g][...] + 1e-9))
    z = jnp.concatenate(cols, axis=1)
    z = jnp.maximum(z, 0.0)
    e = jnp.dot(z, w2_ref[...], preferred_element_type=jnp.float32) + b2_ref[...]
    enc_ref[...] = e
    e = jnp.maximum(e, 0.0)
    log_ref[...] = jnp.dot(e, w3_ref[...], preferred_element_type=jnp.float32) + b3_ref[...]


def _out_proj(hs, e0, n0, e1, n1, W2, b2, W3, b3):
    blk = 1000
    nb = _N // blk
    dspec = pl.BlockSpec((blk, _W), lambda i: (i, 0))
    enc, log = pl.pallas_call(
        _out_proj_body,
        grid=(nb,),
        in_specs=[dspec] * (5 * _G) + [
            pl.BlockSpec((2 * _D, _D), lambda i: (0, 0)),
            pl.BlockSpec((_D,), lambda i: (0,)),
            pl.BlockSpec((_D, 16), lambda i: (0, 0)),
            pl.BlockSpec((16,), lambda i: (0,)),
        ],
        out_specs=[
            pl.BlockSpec((blk, _D), lambda i: (i, 0)),
            pl.BlockSpec((blk, 16), lambda i: (i, 0)),
        ],
        out_shape=[
            jax.ShapeDtypeStruct((_N, _D), jnp.float32),
            jax.ShapeDtypeStruct((_N, 16), jnp.float32),
        ],
    )(*hs, *e0, *n0, *e1, *n1, W2, b2, W3, b3)
    return enc, log


_MESH = plsc.VectorSubcoreMesh(core_axis_name="c", subcore_axis_name="s")
_SC_PARAMS = pltpu.CompilerParams(use_tc_tiling_on_sc=False)


@functools.partial(
    pl.kernel,
    out_type=[
        jax.ShapeDtypeStruct((_ER, _LM), jnp.float32),
        jax.ShapeDtypeStruct((_ER, _LM), jnp.float32),
    ],
    mesh=_MESH,
    compiler_params=_SC_PARAMS,
    scratch_types=[
        pltpu.VMEM((_RC, _LM), jnp.int32),
        pltpu.VMEM((_RC, _LM), jnp.float32),
        pltpu.SemaphoreType.DMA,
    ],
)
def _gather_s(s_hbm, src2, dst2, ssrc2, sdst2, iv, gv, sem):
    cid = lax.axis_index("c")
    sid = lax.axis_index("s")

    def do(idx2, out2):
        def step(k, _):
            c = k * _NS + sid

            @pl.when(c < _NCHUNK)
            def _():
                rb = c * _RC
                pltpu.sync_copy(idx2.at[pl.ds(rb, _RC)], iv)
                for j in range(_RC):
                    pltpu.async_copy(s_hbm.at[iv.at[j]], gv.at[j], sem).wait()
                pltpu.sync_copy(gv, out2.at[pl.ds(rb, _RC)])

            return 0

        lax.fori_loop(0, _KPS, step, 0)

    @pl.when(cid == 0)
    def _():
        do(src2, ssrc2)

    @pl.when(cid == 1)
    def _():
        do(dst2, sdst2)


def _local_idx(dstv, idxv, base_n):
    garb = _HALF + lax.broadcasted_iota(jnp.int32, (16,), 0)
    for j in range(_RC):
        for g in range(_LM // 16):
            d = dstv[j, pl.ds(g * 16, 16)]
            loc = d - base_n
            ok = (loc >= 0) & (loc < _HALF)
            idxv[j, pl.ds(g * 16, 16)] = jnp.where(ok, loc, garb)


def _acc_init(zeros_hbm, acc, sid):
    plsc.subcore_barrier()
    pltpu.sync_copy(zeros_hbm.at[pl.ds(sid * _ZR, _ZR)], acc.at[pl.ds(sid * _ZR, _ZR)])
    plsc.subcore_barrier()


def _acc_flush(acc, out_hbm, base_n, sid):
    plsc.subcore_barrier()
    fb = sid * _FR
    pltpu.sync_copy(acc.at[pl.ds(fb, _FR)], out_hbm.at[pl.ds(base_n + fb, _FR)])

    @pl.when(sid < (_HALF - _NS * _FR) // 8)
    def _():
        tb = _NS * _FR + sid * 8
        pltpu.sync_copy(acc.at[pl.ds(tb, 8)], out_hbm.at[pl.ds(base_n + tb, 8)])


@functools.partial(
    pl.kernel,
    out_type=[jax.ShapeDtypeStruct((_N, _W), jnp.float32) for _ in range(_G)],
    mesh=_MESH,
    compiler_params=_SC_PARAMS,
    scratch_types=[
        pltpu.VMEM((_RC, _LM), jnp.int32),
        pltpu.VMEM((_RC, _LM), jnp.int32),
        pltpu.VMEM((_C, _W), jnp.float32),
        pltpu.VMEM_SHARED((_AROWS, _W), jnp.float32),
    ],
)
def _pass1(ex0, ex1, ex2, ex3, dst2, zeros_hbm,
           out0, out1, out2_, out3, dstv, idxv, exv, acc):
    cid = lax.axis_index("c")
    sid = lax.axis_index("s")
    base_n = cid * _HALF
    exs = [ex0, ex1, ex2, ex3]
    outs = [out0, out1, out2_, out3]
    for g in range(_G):
        _acc_init(zeros_hbm, acc, sid)

        def step(k, _, g=g):
            c = k * _NS + sid

            @pl.when(c < _NCHUNK)
            def _():
                rb = c * _RC
                eb = c * _C
                pltpu.sync_copy(dst2.at[pl.ds(rb, _RC)], dstv)
                pltpu.sync_copy(exs[g].at[pl.ds(eb, _C)], exv)
                _local_idx(dstv, idxv, base_n)
                for j in range(_RC):
                    pltpu.sync_copy(exv.at[pl.ds(j * _LM, _LM)],
                                    acc.at[idxv.at[j]], add=True)

            return 0

        lax.fori_loop(0, _KPS, step, 0)
        _acc_flush(acc, outs[g], base_n, sid)


@functools.partial(
    pl.kernel,
    out_type=[jax.ShapeDtypeStruct((_N, _W), jnp.float32) for _ in range(_G)],
    mesh=_MESH,
    compiler_params=_SC_PARAMS,
    scratch_types=[
        pltpu.VMEM((_RC, _LM), jnp.int32),
        pltpu.VMEM((_RC, _LM), jnp.int32),
        pltpu.VMEM((_RC, _LM), jnp.int32),
        pltpu.VMEM((_C, _W), jnp.float32),
        pltpu.VMEM((_C, _W), jnp.float32),
        pltpu.VMEM_SHARED((_AROWS, _W), jnp.float32),
        pltpu.SemaphoreType.DMA,
    ],
)
def _pass2(ex0, ex1, ex2, ex3, h0, h1, h2, h3, src2, dst2, zeros_hbm,
           out0, out1, out2_, out3, srcv, dstv, idxv, exv, hv, acc, sem):
    cid = lax.axis_index("c")
    sid = lax.axis_index("s")
    base_n = cid * _HALF
    exs = [ex0, ex1, ex2, ex3]
    hs = [h0, h1, h2, h3]
    outs = [out0, out1, out2_, out3]
    for g in range(_G):
        _acc_init(zeros_hbm, acc, sid)

        def step(k, _, g=g):
            c = k * _NS + sid

            @pl.when(c < _NCHUNK)
            def _():
                rb = c * _RC
                eb = c * _C
                pltpu.sync_copy(src2.at[pl.ds(rb, _RC)], srcv)
                pltpu.sync_copy(dst2.at[pl.ds(rb, _RC)], dstv)
                pltpu.sync_copy(exs[g].at[pl.ds(eb, _C)], exv)
                for j in range(_RC):
                    pltpu.async_copy(hs[g].at[srcv.at[j]],
                                     hv.at[pl.ds(j * _LM, _LM)], sem).wait()
                _local_idx(dstv, idxv, base_n)

                def mul_row(r, _):
                    exv[r, pl.ds(0, 16)] = exv[r, pl.ds(0, 16)] * hv[r, pl.ds(0, 16)]
                    return 0

                lax.fori_loop(0, _C, mul_row, 0)
                for j in range(_RC):
                    pltpu.sync_copy(exv.at[pl.ds(j * _LM, _LM)],
                                    acc.at[idxv.at[j]], add=True)

            return 0

        lax.fori_loop(0, _KPS, step, 0)
        _acc_flush(acc, outs[g], base_n, sid)


def _conv(hs, s_n1, src2, dst2, zeros, al_c, ar_c):
    ssrc2, sdst2 = _gather_s(s_n1.reshape(_N), src2, dst2)
    exs = _exk(ssrc2.reshape(_E, 1), sdst2.reshape(_E, 1), al_c, ar_c)
    esums = list(_pass1(*exs, dst2, zeros))
    numers = list(_pass2(*exs, *hs, src2, dst2, zeros))
    return esums, numers


def kernel(feat0, feat1, edge_index, e_feat, W1_0, b1_0, W1_1, b1_1, al, ar, W2, b2, W3, b3):
    src2 = edge_index[0].reshape(_ER, _LM)
    dst2 = edge_index[1].reshape(_ER, _LM)
    zeros = jnp.zeros((_AROWS, _W), jnp.float32)
    hs, s0 = _in_proj(feat0, feat1, W1_0, b1_0, W1_1, b1_1)
    finals = []
    for hd in range(_HEADS):
        xs, sx = hs, s0
        for c in range(_CONVS):
            esums, numers = _conv(xs, sx, src2, dst2, zeros, al[hd, c], ar[hd, c])
            if c < _CONVS - 1:
                xs, sx = _prep(esums, numers)
        finals.append((esums, numers))
    (e0, n0), (e1, n1) = finals
    enc, log = _out_proj(hs, e0, n0, e1, n1, W2, b2, W3, b3)
    return (log, enc)


# speedup vs baseline: 3.3704x; 1.7460x over previous
import functools
import jax
import jax.numpy as jnp
from jax import lax
from jax.experimental import pallas as pl
from jax.experimental.pallas import tpu as pltpu
from jax.experimental.pallas import tpu_sc as plsc

_N0 = 25000
_N1 = 25000
_N = _N0 + _N1
_E = 800000
_D = 64
_G = 4
_W = _D // _G
_HEADS = 2
_CONVS = 2

_NC = 2
_NS = 16
_LM = 80
_RC = 8
_C = _LM * _RC
_ER = _E // _LM
_NCHUNK = _E // _C
_KPS = -(-_NCHUNK // _NS)
_HALF = _N // _NC
_ZR = 1568
_AROWS = _ZR * _NS
_FR = 1560


def _in_proj_body(f0_ref, f1_ref, w0_ref, b0_ref, w1_ref, b1_ref, *outs):
    i = pl.program_id(0)
    nb = pl.num_programs(0)
    half = nb // 2
    h_refs = outs[:_G]
    s_ref = outs[_G]

    def emit(h):
        for g in range(_G):
            h_refs[g][...] = h[:, g * _W:(g + 1) * _W]
        s_ref[...] = jnp.sum(h, axis=1, keepdims=True)

    @pl.when(i < half)
    def _():
        emit(jnp.dot(f0_ref[...], w0_ref[...], preferred_element_type=jnp.float32)
             + b0_ref[...])

    @pl.when(i >= half)
    def _():
        emit(jnp.dot(f1_ref[...], w1_ref[...], preferred_element_type=jnp.float32)
             + b1_ref[...])


def _in_proj(feat0, feat1, W1_0, b1_0, W1_1, b1_1):
    blk = 1000
    nb0 = _N0 // blk
    nb = 2 * nb0

    def f0_idx(i):
        return (jnp.minimum(i, nb0 - 1), 0)

    def f1_idx(i):
        return (jnp.maximum(i - nb0, 0), 0)

    outs = pl.pallas_call(
        _in_proj_body,
        grid=(nb,),
        in_specs=[
            pl.BlockSpec((blk, 128), f0_idx),
            pl.BlockSpec((blk, 128), f1_idx),
            pl.BlockSpec((128, _D), lambda i: (0, 0)),
            pl.BlockSpec((_D,), lambda i: (0,)),
            pl.BlockSpec((128, _D), lambda i: (0, 0)),
            pl.BlockSpec((_D,), lambda i: (0,)),
        ],
        out_specs=[pl.BlockSpec((blk, _W), lambda i: (i, 0)) for _ in range(_G)]
        + [pl.BlockSpec((blk, 1), lambda i: (i, 0))],
        out_shape=[jax.ShapeDtypeStruct((_N, _W), jnp.float32) for _ in range(_G)]
        + [jax.ShapeDtypeStruct((_N, 1), jnp.float32)],
    )(feat0, feat1, W1_0, b1_0, W1_1, b1_1)
    return list(outs[:_G]), outs[_G]


def _exk_body(u_ref, v_ref, al_ref, ar_ref, *ex_refs):
    x = u_ref[...] * al_ref[...] + v_ref[...] * ar_ref[...]
    x = jnp.where(x >= 0.0, x, 0.2 * x)
    ex = jnp.exp(x)
    for g in range(_G):
        ex_refs[g][...] = ex[:, g * _W:(g + 1) * _W]


def _exk(ssrc, sdst, al_c, ar_c):
    blk = 2000
    nb = _E // blk
    exs = pl.pallas_call(
        _exk_body,
        grid=(nb,),
        in_specs=[
            pl.BlockSpec((blk, 1), lambda i: (i, 0)),
            pl.BlockSpec((blk, 1), lambda i: (i, 0)),
            pl.BlockSpec((1, _D), lambda i: (0, 0)),
            pl.BlockSpec((1, _D), lambda i: (0, 0)),
        ],
        out_specs=[pl.BlockSpec((blk, _W), lambda i: (i, 0)) for _ in range(_G)],
        out_shape=[jax.ShapeDtypeStruct((_E, _W), jnp.float32) for _ in range(_G)],
    )(ssrc, sdst, al_c, ar_c)
    return list(exs)


def _prep_body(*refs):
    e_refs = refs[:_G]
    n_refs = refs[_G:2 * _G]
    x_refs = refs[2 * _G:3 * _G]
    s_ref = refs[3 * _G]
    s = None
    for g in range(_G):
        x = n_refs[g][...] / (e_refs[g][...] + 1e-9)
        x_refs[g][...] = x
        r = jnp.sum(x, axis=1, keepdims=True)
        s = r if s is None else s + r
    s_ref[...] = s


def _prep(esums, numers):
    blk = 2000
    nb = _N // blk
    dspec = pl.BlockSpec((blk, _W), lambda i: (i, 0))
    outs = pl.pallas_call(
        _prep_body,
        grid=(nb,),
        in_specs=[dspec] * (2 * _G),
        out_specs=[dspec] * _G + [pl.BlockSpec((blk, 1), lambda i: (i, 0))],
        out_shape=[jax.ShapeDtypeStruct((_N, _W), jnp.float32) for _ in range(_G)]
        + [jax.ShapeDtypeStruct((_N, 1), jnp.float32)],
    )(*esums, *numers)
    return list(outs[:_G]), outs[_G]


def _out_proj_body(*refs):
    h_refs = refs[:_G]
    e0_refs = refs[_G:2 * _G]
    n0_refs = refs[2 * _G:3 * _G]
    e1_refs = refs[3 * _G:4 * _G]
    n1_refs = refs[4 * _G:5 * _G]
    w2_ref, b2_ref, w3_ref, b3_ref, enc_ref, log_ref = refs[5 * _G:]
    cols = []
    for g in range(_G):
        cols.append(h_refs[g][...] + n0_refs[g][...] / (e0_refs[g][...] + 1e-9))
    for g in range(_G):
        cols.append(h_refs[g][...] + n1_refs[g][...] / (e1_refs[g][...] + 1e-9))
    z = jnp.concatenate(cols, axis=1)
    z = jnp.maximum(z, 0.0)
    e = jnp.dot(z, w2_ref[...], preferred_element_type=jnp.float32) + b2_ref[...]
    enc_ref[...] = e
    e = jnp.maximum(e, 0.0)
    log_ref[...] = jnp.dot(e, w3_ref[...], preferred_element_type=jnp.float32) + b3_ref[...]


def _out_proj(hs, e0, n0, e1, n1, W2, b2, W3, b3):
    blk = 1000
    nb = _N // blk
    dspec = pl.BlockSpec((blk, _W), lambda i: (i, 0))
    enc, log = pl.pallas_call(
        _out_proj_body,
        grid=(nb,),
        in_specs=[dspec] * (5 * _G) + [
            pl.BlockSpec((2 * _D, _D), lambda i: (0, 0)),
            pl.BlockSpec((_D,), lambda i: (0,)),
            pl.BlockSpec((_D, 16), lambda i: (0, 0)),
            pl.BlockSpec((16,), lambda i: (0,)),
        ],
        out_specs=[
            pl.BlockSpec((blk, _D), lambda i: (i, 0)),
            pl.BlockSpec((blk, 16), lambda i: (i, 0)),
        ],
        out_shape=[
            jax.ShapeDtypeStruct((_N, _D), jnp.float32),
            jax.ShapeDtypeStruct((_N, 16), jnp.float32),
        ],
    )(*hs, *e0, *n0, *e1, *n1, W2, b2, W3, b3)
    return enc, log


_MESH = plsc.VectorSubcoreMesh(core_axis_name="c", subcore_axis_name="s")
_SC_PARAMS = pltpu.CompilerParams(use_tc_tiling_on_sc=False)


@functools.partial(
    pl.kernel,
    out_type=[
        jax.ShapeDtypeStruct((_ER, _LM), jnp.float32),
        jax.ShapeDtypeStruct((_ER, _LM), jnp.float32),
    ],
    mesh=_MESH,
    compiler_params=_SC_PARAMS,
    scratch_types=[
        pltpu.VMEM((_RC, _LM), jnp.int32),
        pltpu.VMEM((_RC, _LM), jnp.float32),
        pltpu.SemaphoreType.DMA,
    ],
)
def _gather_s(s_hbm, src2, dst2, ssrc2, sdst2, iv, gv, sem):
    cid = lax.axis_index("c")
    sid = lax.axis_index("s")

    def do(idx2, out2):
        def step(k, _):
            c = k * _NS + sid

            @pl.when(c < _NCHUNK)
            def _():
                rb = c * _RC
                pltpu.sync_copy(idx2.at[pl.ds(rb, _RC)], iv)
                for j in range(_RC):
                    pltpu.async_copy(s_hbm.at[iv.at[j]], gv.at[j], sem).wait()
                pltpu.sync_copy(gv, out2.at[pl.ds(rb, _RC)])

            return 0

        lax.fori_loop(0, _KPS, step, 0)

    @pl.when(cid == 0)
    def _():
        do(src2, ssrc2)

    @pl.when(cid == 1)
    def _():
        do(dst2, sdst2)


def _local_idx(dstv, idxv, base_n):
    garb = _HALF + lax.broadcasted_iota(jnp.int32, (16,), 0)
    for j in range(_RC):
        for g in range(_LM // 16):
            d = dstv[j, pl.ds(g * 16, 16)]
            loc = d - base_n
            ok = (loc >= 0) & (loc < _HALF)
            idxv[j, pl.ds(g * 16, 16)] = jnp.where(ok, loc, garb)


def _acc_init(zeros_hbm, acc, sid):
    plsc.subcore_barrier()
    pltpu.sync_copy(zeros_hbm.at[pl.ds(sid * _ZR, _ZR)], acc.at[pl.ds(sid * _ZR, _ZR)])
    plsc.subcore_barrier()


def _acc_flush(acc, out_hbm, base_n, sid):
    plsc.subcore_barrier()
    fb = sid * _FR
    pltpu.sync_copy(acc.at[pl.ds(fb, _FR)], out_hbm.at[pl.ds(base_n + fb, _FR)])

    @pl.when(sid < (_HALF - _NS * _FR) // 8)
    def _():
        tb = _NS * _FR + sid * 8
        pltpu.sync_copy(acc.at[pl.ds(tb, 8)], out_hbm.at[pl.ds(base_n + tb, 8)])


_NB = 3
_KT = 27


def _idx_chunk(c):
    return jnp.minimum(c, _NCHUNK - 1)


def _compute_idx(dstv, idxv, base_n, c):
    garb = _HALF + lax.broadcasted_iota(jnp.int32, (16,), 0)
    shift = base_n - jnp.where(c < _NCHUNK, 0, 4 * _N)
    for j in range(_RC):
        for g in range(_LM // 16):
            d = dstv[j, pl.ds(g * 16, 16)]
            loc = d - shift
            ok = (loc >= 0) & (loc < _HALF)
            idxv[j, pl.ds(g * 16, 16)] = jnp.where(ok, loc, garb)


def _fill_garbage(idxv):
    garb = _HALF + lax.broadcasted_iota(jnp.int32, (16,), 0)
    for j in range(_RC):
        for g in range(_LM // 16):
            idxv[j, pl.ds(g * 16, 16)] = garb


def _fire_scatter(exv, idxv, acc, sem):
    for j in range(_RC):
        pltpu.async_copy(exv.at[pl.ds(j * _LM, _LM)], acc.at[idxv.at[j]], sem,
                         add=True)


def _drain_scatter(exv, idxv, acc, sem):
    for j in range(_RC):
        pltpu.make_async_copy(exv.at[pl.ds(j * _LM, _LM)], acc.at[idxv.at[j]],
                              sem).wait()


def _issue_linear(ex_g, dst2, c, dstv, exv, sem, src2=None, srcv=None):
    rb = c * _RC
    eb = c * _C
    pltpu.async_copy(dst2.at[pl.ds(rb, _RC)], dstv, sem)
    pltpu.async_copy(ex_g.at[pl.ds(eb, _C)], exv, sem)
    if src2 is not None:
        pltpu.async_copy(src2.at[pl.ds(rb, _RC)], srcv, sem)


def _wait_linear(ex_g, dst2, dstv, exv, sem, src2=None, srcv=None):
    pltpu.make_async_copy(dst2.at[pl.ds(0, _RC)], dstv, sem).wait()
    pltpu.make_async_copy(ex_g.at[pl.ds(0, _C)], exv, sem).wait()
    if src2 is not None:
        pltpu.make_async_copy(src2.at[pl.ds(0, _RC)], srcv, sem).wait()


@functools.partial(
    pl.kernel,
    out_type=[jax.ShapeDtypeStruct((_N, _W), jnp.float32) for _ in range(_G)],
    mesh=_MESH,
    compiler_params=_SC_PARAMS,
    scratch_types=(
        [pltpu.VMEM((_RC, _LM), jnp.int32) for _ in range(2 * _NB)]
        + [pltpu.VMEM((_C, _W), jnp.float32) for _ in range(_NB)]
        + [pltpu.VMEM_SHARED((_AROWS, _W), jnp.float32)]
        + [pltpu.SemaphoreType.DMA for _ in range(2 * _NB)]
    ),
)
def _pass1(ex0, ex1, ex2, ex3, dst2, zeros_hbm, out0, out1, out2_, out3, *scr):
    dstv = scr[0:_NB]
    idxv = scr[_NB:2 * _NB]
    exv = scr[2 * _NB:3 * _NB]
    acc = scr[3 * _NB]
    sem_l = scr[3 * _NB + 1:3 * _NB + 1 + _NB]
    sem_sc = scr[3 * _NB + 1 + _NB:3 * _NB + 1 + 2 * _NB]
    cid = lax.axis_index("c")
    sid = lax.axis_index("s")
    base_n = cid * _HALF
    exs = [ex0, ex1, ex2, ex3]
    outs = [out0, out1, out2_, out3]
    for g in range(_G):
        exg = exs[g]
        _acc_init(zeros_hbm, acc, sid)
        _fill_garbage(idxv[2])
        _fire_scatter(exv[2], idxv[2], acc, sem_sc[2])
        _issue_linear(exg, dst2, _idx_chunk(sid), dstv[0], exv[0], sem_l[0])
        _wait_linear(exg, dst2, dstv[0], exv[0], sem_l[0])
        _issue_linear(exg, dst2, _idx_chunk(16 + sid), dstv[1], exv[1], sem_l[1])

        def it(i, _, exg=exg):
            for t in range(3):
                k = i * 3 + t
                b = t
                b1 = (t + 1) % 3
                b2 = (t + 2) % 3
                c = k * _NS + sid
                _drain_scatter(exv[b2], idxv[b2], acc, sem_sc[b2])
                _wait_linear(exg, dst2, dstv[b1], exv[b1], sem_l[b1])
                _issue_linear(exg, dst2, _idx_chunk((k + 2) * _NS + sid),
                              dstv[b2], exv[b2], sem_l[b2])
                _compute_idx(dstv[b], idxv[b], base_n, c)
                _fire_scatter(exv[b], idxv[b], acc, sem_sc[b])
            return 0

        lax.fori_loop(0, _KT, it, 0)
        _drain_scatter(exv[2], idxv[2], acc, sem_sc[2])
        _wait_linear(exg, dst2, dstv[1], exv[1], sem_l[1])
        _acc_flush(acc, outs[g], base_n, sid)


@functools.partial(
    pl.kernel,
    out_type=[jax.ShapeDtypeStruct((_N, _W), jnp.float32) for _ in range(_G)],
    mesh=_MESH,
    compiler_params=_SC_PARAMS,
    scratch_types=(
        [pltpu.VMEM((_RC, _LM), jnp.int32) for _ in range(3 * _NB)]
        + [pltpu.VMEM((_C, _W), jnp.float32) for _ in range(2 * _NB)]
        + [pltpu.VMEM_SHARED((_AROWS, _W), jnp.float32)]
        + [pltpu.SemaphoreType.DMA for _ in range(3 * _NB)]
    ),
)
def _pass2(ex0, ex1, ex2, ex3, h0, h1, h2, h3, src2, dst2, zeros_hbm,
           out0, out1, out2_, out3, *scr):
    srcv = scr[0:_NB]
    dstv = scr[_NB:2 * _NB]
    idxv = scr[2 * _NB:3 * _NB]
    exv = scr[3 * _NB:4 * _NB]
    hv = scr[4 * _NB:5 * _NB]
    acc = scr[5 * _NB]
    sem_l = scr[5 * _NB + 1:5 * _NB + 1 + _NB]
    sem_sc = scr[5 * _NB + 1 + _NB:5 * _NB + 1 + 2 * _NB]
    sem_h = scr[5 * _NB + 1 + 2 * _NB:5 * _NB + 1 + 3 * _NB]
    cid = lax.axis_index("c")
    sid = lax.axis_index("s")
    base_n = cid * _HALF
    exs = [ex0, ex1, ex2, ex3]
    hs = [h0, h1, h2, h3]
    outs = [out0, out1, out2_, out3]

    def fire_h(hg, sv, hb, sem):
        for j in range(_RC):
            pltpu.async_copy(hg.at[sv.at[j]], hb.at[pl.ds(j * _LM, _LM)], sem)

    def drain_h(hg, sv, hb, sem):
        for j in range(_RC):
            pltpu.make_async_copy(hg.at[sv.at[j]], hb.at[pl.ds(j * _LM, _LM)],
                                  sem).wait()

    for g in range(_G):
        exg = exs[g]
        hg = hs[g]
        _acc_init(zeros_hbm, acc, sid)
        _fill_garbage(idxv[2])
        _fire_scatter(exv[2], idxv[2], acc, sem_sc[2])
        _issue_linear(exg, dst2, _idx_chunk(sid), dstv[0], exv[0], sem_l[0],
                      src2, srcv[0])
        _wait_linear(exg, dst2, dstv[0], exv[0], sem_l[0], src2, srcv[0])
        _issue_linear(exg, dst2, _idx_chunk(16 + sid), dstv[1], exv[1],
                      sem_l[1], src2, srcv[1])
        fire_h(hg, srcv[0], hv[0], sem_h[0])

        def it(i, _, exg=exg, hg=hg):
            for t in range(3):
                k = i * 3 + t
                b = t
                b1 = (t + 1) % 3
                b2 = (t + 2) % 3
                c = k * _NS + sid
                _drain_scatter(exv[b2], idxv[b2], acc, sem_sc[b2])
                _wait_linear(exg, dst2, dstv[b1], exv[b1], sem_l[b1],
                             src2, srcv[b1])
                _issue_linear(exg, dst2, _idx_chunk((k + 2) * _NS + sid),
                              dstv[b2], exv[b2], sem_l[b2], src2, srcv[b2])
                fire_h(hg, srcv[b1], hv[b1], sem_h[b1])
                drain_h(hg, srcv[b], hv[b], sem_h[b])
                _compute_idx(dstv[b], idxv[b], base_n, c)

                def mul_row(r, _):
                    exv[b][r, pl.ds(0, 16)] = (exv[b][r, pl.ds(0, 16)]
                                               * hv[b][r, pl.ds(0, 16)])
                    return 0

                lax.fori_loop(0, _C, mul_row, 0)
                _fire_scatter(exv[b], idxv[b], acc, sem_sc[b])
            return 0

        lax.fori_loop(0, _KT, it, 0)
        _drain_scatter(exv[2], idxv[2], acc, sem_sc[2])
        _wait_linear(exg, dst2, dstv[1], exv[1], sem_l[1], src2, srcv[1])
        drain_h(hg, srcv[0], hv[0], sem_h[0])
        _acc_flush(acc, outs[g], base_n, sid)


def _conv(hs, s_n1, src2, dst2, zeros, al_c, ar_c):
    ssrc2, sdst2 = _gather_s(s_n1.reshape(_N), src2, dst2)
    exs = _exk(ssrc2.reshape(_E, 1), sdst2.reshape(_E, 1), al_c, ar_c)
    esums = list(_pass1(*exs, dst2, zeros))
    numers = list(_pass2(*exs, *hs, src2, dst2, zeros))
    return esums, numers


def kernel(feat0, feat1, edge_index, e_feat, W1_0, b1_0, W1_1, b1_1, al, ar, W2, b2, W3, b3):
    src2 = edge_index[0].reshape(_ER, _LM)
    dst2 = edge_index[1].reshape(_ER, _LM)
    zeros = jnp.zeros((_AROWS, _W), jnp.float32)
    hs, s0 = _in_proj(feat0, feat1, W1_0, b1_0, W1_1, b1_1)
    finals = []
    for hd in range(_HEADS):
        xs, sx = hs, s0
        for c in range(_CONVS):
            esums, numers = _conv(xs, sx, src2, dst2, zeros, al[hd, c], ar[hd, c])
            if c < _CONVS - 1:
                xs, sx = _prep(esums, numers)
        finals.append((esums, numers))
    (e0, n0), (e1, n1) = finals
    enc, log = _out_proj(hs, e0, n0, e1, n1, W2, b2, W3, b3)
    return (log, enc)


# speedup vs baseline: 3.8144x; 1.1317x over previous
import functools
import jax
import jax.numpy as jnp
from jax import lax
from jax.experimental import pallas as pl
from jax.experimental.pallas import tpu as pltpu
from jax.experimental.pallas import tpu_sc as plsc

_N0 = 25000
_N1 = 25000
_N = _N0 + _N1
_E = 800000
_D = 64
_G = 4
_W = _D // _G
_HEADS = 2
_CONVS = 2

_NC = 2
_NS = 16
_LM = 80
_RC = 8
_C = _LM * _RC
_ER = _E // _LM
_NCHUNK = _E // _C
_KPS = -(-_NCHUNK // _NS)
_HALF = _N // _NC
_ZR = 1568
_AROWS = _ZR * _NS
_FR = 1560


def _in_proj_body(f0_ref, f1_ref, w0_ref, b0_ref, w1_ref, b1_ref, *outs):
    i = pl.program_id(0)
    nb = pl.num_programs(0)
    half = nb // 2
    h_refs = outs[:_G]
    s_ref = outs[_G]

    def emit(h):
        for g in range(_G):
            h_refs[g][...] = h[:, g * _W:(g + 1) * _W]
        s_ref[...] = jnp.sum(h, axis=1, keepdims=True)

    @pl.when(i < half)
    def _():
        emit(jnp.dot(f0_ref[...], w0_ref[...], preferred_element_type=jnp.float32)
             + b0_ref[...])

    @pl.when(i >= half)
    def _():
        emit(jnp.dot(f1_ref[...], w1_ref[...], preferred_element_type=jnp.float32)
             + b1_ref[...])


def _in_proj(feat0, feat1, W1_0, b1_0, W1_1, b1_1):
    blk = 1000
    nb0 = _N0 // blk
    nb = 2 * nb0

    def f0_idx(i):
        return (jnp.minimum(i, nb0 - 1), 0)

    def f1_idx(i):
        return (jnp.maximum(i - nb0, 0), 0)

    outs = pl.pallas_call(
        _in_proj_body,
        grid=(nb,),
        in_specs=[
            pl.BlockSpec((blk, 128), f0_idx),
            pl.BlockSpec((blk, 128), f1_idx),
            pl.BlockSpec((128, _D), lambda i: (0, 0)),
            pl.BlockSpec((_D,), lambda i: (0,)),
            pl.BlockSpec((128, _D), lambda i: (0, 0)),
            pl.BlockSpec((_D,), lambda i: (0,)),
        ],
        out_specs=[pl.BlockSpec((blk, _W), lambda i: (i, 0)) for _ in range(_G)]
        + [pl.BlockSpec((blk, 1), lambda i: (i, 0))],
        out_shape=[jax.ShapeDtypeStruct((_N, _W), jnp.float32) for _ in range(_G)]
        + [jax.ShapeDtypeStruct((_N, 1), jnp.float32)],
    )(feat0, feat1, W1_0, b1_0, W1_1, b1_1)
    return list(outs[:_G]), outs[_G]


def _exk_body(u_ref, v_ref, al_ref, ar_ref, *ex_refs):
    x = u_ref[...] * al_ref[...] + v_ref[...] * ar_ref[...]
    x = jnp.where(x >= 0.0, x, 0.2 * x)
    ex = jnp.exp(x)
    for g in range(_G):
        ex_refs[g][...] = ex[:, g * _W:(g + 1) * _W]


def _exk(ssrc, sdst, al_c, ar_c):
    blk = 2000
    nb = _E // blk
    exs = pl.pallas_call(
        _exk_body,
        grid=(nb,),
        in_specs=[
            pl.BlockSpec((blk, 1), lambda i: (i, 0)),
            pl.BlockSpec((blk, 1), lambda i: (i, 0)),
            pl.BlockSpec((1, _D), lambda i: (0, 0)),
            pl.BlockSpec((1, _D), lambda i: (0, 0)),
        ],
        out_specs=[pl.BlockSpec((blk, _W), lambda i: (i, 0)) for _ in range(_G)],
        out_shape=[jax.ShapeDtypeStruct((_E, _W), jnp.float32) for _ in range(_G)],
    )(ssrc, sdst, al_c, ar_c)
    return list(exs)


def _prep_body(*refs):
    e_refs = refs[:_G]
    n_refs = refs[_G:2 * _G]
    x_refs = refs[2 * _G:3 * _G]
    s_ref = refs[3 * _G]
    s = None
    for g in range(_G):
        x = n_refs[g][...] / (e_refs[g][...] + 1e-9)
        x_refs[g][...] = x
        r = jnp.sum(x, axis=1, keepdims=True)
        s = r if s is None else s + r
    s_ref[...] = s


def _prep(esums, numers):
    blk = 2000
    nb = _N // blk
    dspec = pl.BlockSpec((blk, _W), lambda i: (i, 0))
    outs = pl.pallas_call(
        _prep_body,
        grid=(nb,),
        in_specs=[dspec] * (2 * _G),
        out_specs=[dspec] * _G + [pl.BlockSpec((blk, 1), lambda i: (i, 0))],
        out_shape=[jax.ShapeDtypeStruct((_N, _W), jnp.float32) for _ in range(_G)]
        + [jax.ShapeDtypeStruct((_N, 1), jnp.float32)],
    )(*esums, *numers)
    return list(outs[:_G]), outs[_G]


def _out_proj_body(*refs):
    h_refs = refs[:_G]
    e0_refs = refs[_G:2 * _G]
    n0_refs = refs[2 * _G:3 * _G]
    e1_refs = refs[3 * _G:4 * _G]
    n1_refs = refs[4 * _G:5 * _G]
    w2_ref, b2_ref, w3_ref, b3_ref, enc_ref, log_ref = refs[5 * _G:]
    cols = []
    for g in range(_G):
        cols.append(h_refs[g][...] + n0_refs[g][...] / (e0_refs[g][...] + 1e-9))
    for g in range(_G):
        cols.append(h_refs[g][...] + n1_refs[g][...] / (e1_refs[g][...] + 1e-9))
    z = jnp.concatenate(cols, axis=1)
    z = jnp.maximum(z, 0.0)
    e = jnp.dot(z, w2_ref[...], preferred_element_type=jnp.float32) + b2_ref[...]
    enc_ref[...] = e
    e = jnp.maximum(e, 0.0)
    log_ref[...] = jnp.dot(e, w3_ref[...], preferred_element_type=jnp.float32) + b3_ref[...]


def _out_proj(hs, e0, n0, e1, n1, W2, b2, W3, b3):
    blk = 1000
    nb = _N // blk
    dspec = pl.BlockSpec((blk, _W), lambda i: (i, 0))
    enc, log = pl.pallas_call(
        _out_proj_body,
        grid=(nb,),
        in_specs=[dspec] * (5 * _G) + [
            pl.BlockSpec((2 * _D, _D), lambda i: (0, 0)),
            pl.BlockSpec((_D,), lambda i: (0,)),
            pl.BlockSpec((_D, 16), lambda i: (0, 0)),
            pl.BlockSpec((16,), lambda i: (0,)),
        ],
        out_specs=[
            pl.BlockSpec((blk, _D), lambda i: (i, 0)),
            pl.BlockSpec((blk, 16), lambda i: (i, 0)),
        ],
        out_shape=[
            jax.ShapeDtypeStruct((_N, _D), jnp.float32),
            jax.ShapeDtypeStruct((_N, 16), jnp.float32),
        ],
    )(*hs, *e0, *n0, *e1, *n1, W2, b2, W3, b3)
    return enc, log


_MESH = plsc.VectorSubcoreMesh(core_axis_name="c", subcore_axis_name="s")
_SC_PARAMS = pltpu.CompilerParams(use_tc_tiling_on_sc=False)


@functools.partial(
    pl.kernel,
    out_type=[
        jax.ShapeDtypeStruct((_ER, _LM), jnp.float32),
        jax.ShapeDtypeStruct((_ER, _LM), jnp.float32),
    ],
    mesh=_MESH,
    compiler_params=_SC_PARAMS,
    scratch_types=(
        [pltpu.VMEM((_RC, _LM), jnp.int32) for _ in range(3)]
        + [pltpu.VMEM((_RC, _LM), jnp.float32) for _ in range(3)]
        + [pltpu.SemaphoreType.DMA for _ in range(6)]
    ),
)
def _gather_s(s_hbm, src2, dst2, ssrc2, sdst2, *scr):
    iv = scr[0:3]
    gv = scr[3:6]
    sem_l = scr[6:9]
    sem_g = scr[9:12]
    cid = lax.axis_index("c")
    sid = lax.axis_index("s")

    def do(idx2, out2):
        def issue_iv(c, b):
            pltpu.async_copy(idx2.at[pl.ds(c * _RC, _RC)], iv[b], sem_l[b])

        def wait_iv(b):
            pltpu.make_async_copy(idx2.at[pl.ds(0, _RC)], iv[b], sem_l[b]).wait()

        def fire_g(b):
            for j in range(_RC):
                pltpu.async_copy(s_hbm.at[iv[b].at[j]], gv[b].at[j], sem_g[b])

        def drain_g(b):
            for j in range(_RC):
                pltpu.make_async_copy(s_hbm.at[iv[b].at[j]], gv[b].at[j],
                                      sem_g[b]).wait()

        issue_iv(_idx_chunk(sid), 0)
        wait_iv(0)
        issue_iv(_idx_chunk(_NS + sid), 1)
        fire_g(0)

        def it(i, _):
            for t in range(3):
                k = i * 3 + t
                b = t
                b1 = (t + 1) % 3
                b2 = (t + 2) % 3
                c = _idx_chunk(k * _NS + sid)
                wait_iv(b1)
                issue_iv(_idx_chunk((k + 2) * _NS + sid), b2)
                fire_g(b1)
                drain_g(b)
                pltpu.sync_copy(gv[b], out2.at[pl.ds(c * _RC, _RC)])
            return 0

        lax.fori_loop(0, _KT, it, 0)
        wait_iv(1)
        drain_g(0)

    @pl.when(cid == 0)
    def _():
        do(src2, ssrc2)

    @pl.when(cid == 1)
    def _():
        do(dst2, sdst2)


def _acc_init(zeros_hbm, acc, sid):
    plsc.subcore_barrier()
    pltpu.sync_copy(zeros_hbm.at[pl.ds(sid * _ZR, _ZR)], acc.at[pl.ds(sid * _ZR, _ZR)])
    plsc.subcore_barrier()


def _acc_flush(acc, out_hbm, base_n, sid):
    plsc.subcore_barrier()
    fb = sid * _FR
    pltpu.sync_copy(acc.at[pl.ds(fb, _FR)], out_hbm.at[pl.ds(base_n + fb, _FR)])

    @pl.when(sid < (_HALF - _NS * _FR) // 8)
    def _():
        tb = _NS * _FR + sid * 8
        pltpu.sync_copy(acc.at[pl.ds(tb, 8)], out_hbm.at[pl.ds(base_n + tb, 8)])


_NB = 3
_KT = 27


def _idx_chunk(c):
    return jnp.minimum(c, _NCHUNK - 1)


def _compute_idx(dstv, idxv, base_n, c):
    garb = _HALF + lax.broadcasted_iota(jnp.int32, (16,), 0)
    shift = base_n - jnp.where(c < _NCHUNK, 0, 4 * _N)
    for j in range(_RC):
        for g in range(_LM // 16):
            d = dstv[j, pl.ds(g * 16, 16)]
            loc = d - shift
            ok = (loc >= 0) & (loc < _HALF)
            idxv[j, pl.ds(g * 16, 16)] = jnp.where(ok, loc, garb)


def _fill_garbage(idxv):
    garb = _HALF + lax.broadcasted_iota(jnp.int32, (16,), 0)
    for j in range(_RC):
        for g in range(_LM // 16):
            idxv[j, pl.ds(g * 16, 16)] = garb


def _fire_scatter(exv, idxv, acc, sem):
    for j in range(_RC):
        pltpu.async_copy(exv.at[pl.ds(j * _LM, _LM)], acc.at[idxv.at[j]], sem,
                         add=True)


def _drain_scatter(exv, idxv, acc, sem):
    for j in range(_RC):
        pltpu.make_async_copy(exv.at[pl.ds(j * _LM, _LM)], acc.at[idxv.at[j]],
                              sem).wait()


def _issue_linear(ex_g, dst2, c, dstv, exv, sem, src2=None, srcv=None):
    rb = c * _RC
    eb = c * _C
    pltpu.async_copy(dst2.at[pl.ds(rb, _RC)], dstv, sem)
    pltpu.async_copy(ex_g.at[pl.ds(eb, _C)], exv, sem)
    if src2 is not None:
        pltpu.async_copy(src2.at[pl.ds(rb, _RC)], srcv, sem)


def _wait_linear(ex_g, dst2, dstv, exv, sem, src2=None, srcv=None):
    pltpu.make_async_copy(dst2.at[pl.ds(0, _RC)], dstv, sem).wait()
    pltpu.make_async_copy(ex_g.at[pl.ds(0, _C)], exv, sem).wait()
    if src2 is not None:
        pltpu.make_async_copy(src2.at[pl.ds(0, _RC)], srcv, sem).wait()


@functools.partial(
    pl.kernel,
    out_type=[jax.ShapeDtypeStruct((_N, _W), jnp.float32) for _ in range(_G)],
    mesh=_MESH,
    compiler_params=_SC_PARAMS,
    scratch_types=(
        [pltpu.VMEM((_RC, _LM), jnp.int32) for _ in range(2 * _NB)]
        + [pltpu.VMEM((_C, _W), jnp.float32) for _ in range(_NB)]
        + [pltpu.VMEM_SHARED((_AROWS, _W), jnp.float32)]
        + [pltpu.SemaphoreType.DMA for _ in range(2 * _NB)]
    ),
)
def _pass1(ex0, ex1, ex2, ex3, dst2, zeros_hbm, out0, out1, out2_, out3, *scr):
    dstv = scr[0:_NB]
    idxv = scr[_NB:2 * _NB]
    exv = scr[2 * _NB:3 * _NB]
    acc = scr[3 * _NB]
    sem_l = scr[3 * _NB + 1:3 * _NB + 1 + _NB]
    sem_sc = scr[3 * _NB + 1 + _NB:3 * _NB + 1 + 2 * _NB]
    cid = lax.axis_index("c")
    sid = lax.axis_index("s")
    base_n = cid * _HALF
    exs = [ex0, ex1, ex2, ex3]
    outs = [out0, out1, out2_, out3]
    for g in range(_G):
        exg = exs[g]
        _acc_init(zeros_hbm, acc, sid)
        _fill_garbage(idxv[2])
        _fire_scatter(exv[2], idxv[2], acc, sem_sc[2])
        _issue_linear(exg, dst2, _idx_chunk(sid), dstv[0], exv[0], sem_l[0])
        _wait_linear(exg, dst2, dstv[0], exv[0], sem_l[0])
        _issue_linear(exg, dst2, _idx_chunk(16 + sid), dstv[1], exv[1], sem_l[1])

        def it(i, _, exg=exg):
            for t in range(3):
                k = i * 3 + t
                b = t
                b1 = (t + 1) % 3
                b2 = (t + 2) % 3
                c = k * _NS + sid
                _drain_scatter(exv[b2], idxv[b2], acc, sem_sc[b2])
                _wait_linear(exg, dst2, dstv[b1], exv[b1], sem_l[b1])
                _issue_linear(exg, dst2, _idx_chunk((k + 2) * _NS + sid),
                              dstv[b2], exv[b2], sem_l[b2])
                _compute_idx(dstv[b], idxv[b], base_n, c)
                _fire_scatter(exv[b], idxv[b], acc, sem_sc[b])
            return 0

        lax.fori_loop(0, _KT, it, 0)
        _drain_scatter(exv[2], idxv[2], acc, sem_sc[2])
        _wait_linear(exg, dst2, dstv[1], exv[1], sem_l[1])
        _acc_flush(acc, outs[g], base_n, sid)


@functools.partial(
    pl.kernel,
    out_type=[jax.ShapeDtypeStruct((_N, _W), jnp.float32) for _ in range(_G)],
    mesh=_MESH,
    compiler_params=_SC_PARAMS,
    scratch_types=(
        [pltpu.VMEM((_RC, _LM), jnp.int32) for _ in range(3 * _NB)]
        + [pltpu.VMEM((_C, _W), jnp.float32) for _ in range(2 * _NB)]
        + [pltpu.VMEM_SHARED((_AROWS, _W), jnp.float32)]
        + [pltpu.SemaphoreType.DMA for _ in range(3 * _NB)]
    ),
)
def _pass2(ex0, ex1, ex2, ex3, h0, h1, h2, h3, src2, dst2, zeros_hbm,
           out0, out1, out2_, out3, *scr):
    srcv = scr[0:_NB]
    dstv = scr[_NB:2 * _NB]
    idxv = scr[2 * _NB:3 * _NB]
    exv = scr[3 * _NB:4 * _NB]
    hv = scr[4 * _NB:5 * _NB]
    acc = scr[5 * _NB]
    sem_l = scr[5 * _NB + 1:5 * _NB + 1 + _NB]
    sem_sc = scr[5 * _NB + 1 + _NB:5 * _NB + 1 + 2 * _NB]
    sem_h = scr[5 * _NB + 1 + 2 * _NB:5 * _NB + 1 + 3 * _NB]
    cid = lax.axis_index("c")
    sid = lax.axis_index("s")
    base_n = cid * _HALF
    exs = [ex0, ex1, ex2, ex3]
    hs = [h0, h1, h2, h3]
    outs = [out0, out1, out2_, out3]

    def fire_h(hg, sv, hb, sem):
        for j in range(_RC):
            pltpu.async_copy(hg.at[sv.at[j]], hb.at[pl.ds(j * _LM, _LM)], sem)

    def drain_h(hg, sv, hb, sem):
        for j in range(_RC):
            pltpu.make_async_copy(hg.at[sv.at[j]], hb.at[pl.ds(j * _LM, _LM)],
                                  sem).wait()

    for g in range(_G):
        exg = exs[g]
        hg = hs[g]
        _acc_init(zeros_hbm, acc, sid)
        _fill_garbage(idxv[2])
        _fire_scatter(exv[2], idxv[2], acc, sem_sc[2])
        _issue_linear(exg, dst2, _idx_chunk(sid), dstv[0], exv[0], sem_l[0],
                      src2, srcv[0])
        _wait_linear(exg, dst2, dstv[0], exv[0], sem_l[0], src2, srcv[0])
        _issue_linear(exg, dst2, _idx_chunk(16 + sid), dstv[1], exv[1],
                      sem_l[1], src2, srcv[1])
        fire_h(hg, srcv[0], hv[0], sem_h[0])

        def it(i, _, exg=exg, hg=hg):
            for t in range(3):
                k = i * 3 + t
                b = t
                b1 = (t + 1) % 3
                b2 = (t + 2) % 3
                c = k * _NS + sid
                _drain_scatter(exv[b2], idxv[b2], acc, sem_sc[b2])
                _wait_linear(exg, dst2, dstv[b1], exv[b1], sem_l[b1],
                             src2, srcv[b1])
                _issue_linear(exg, dst2, _idx_chunk((k + 2) * _NS + sid),
                              dstv[b2], exv[b2], sem_l[b2], src2, srcv[b2])
                fire_h(hg, srcv[b1], hv[b1], sem_h[b1])
                drain_h(hg, srcv[b], hv[b], sem_h[b])
                _compute_idx(dstv[b], idxv[b], base_n, c)

                def mul_row(r8, _):
                    for u in range(8):
                        r = r8 * 8 + u
                        exv[b][r, pl.ds(0, 16)] = (exv[b][r, pl.ds(0, 16)]
                                                   * hv[b][r, pl.ds(0, 16)])
                    return 0

                lax.fori_loop(0, _C // 8, mul_row, 0)
                _fire_scatter(exv[b], idxv[b], acc, sem_sc[b])
            return 0

        lax.fori_loop(0, _KT, it, 0)
        _drain_scatter(exv[2], idxv[2], acc, sem_sc[2])
        _wait_linear(exg, dst2, dstv[1], exv[1], sem_l[1], src2, srcv[1])
        drain_h(hg, srcv[0], hv[0], sem_h[0])
        _acc_flush(acc, outs[g], base_n, sid)


def kernel(feat0, feat1, edge_index, e_feat, W1_0, b1_0, W1_1, b1_1, al, ar, W2, b2, W3, b3):
    src2 = edge_index[0].reshape(_ER, _LM)
    dst2 = edge_index[1].reshape(_ER, _LM)
    zeros = jnp.zeros((_AROWS, _W), jnp.float32)
    hs, s0 = _in_proj(feat0, feat1, W1_0, b1_0, W1_1, b1_1)

    def gath(s_n1):
        u2, v2 = _gather_s(s_n1.reshape(_N), src2, dst2)
        return u2.reshape(_E, 1), v2.reshape(_E, 1)

    u0, v0 = gath(s0)
    exA = _exk(u0, v0, al[0, 0], ar[0, 0])
    exB = _exk(u0, v0, al[1, 0], ar[1, 0])

    e0a = list(_pass1(*exA, dst2, zeros))
    n0a = list(_pass2(*exA, *hs, src2, dst2, zeros))
    x0, sx0 = _prep(e0a, n0a)
    uA, vA = gath(sx0)

    e1a = list(_pass1(*exB, dst2, zeros))
    exA1 = _exk(uA, vA, al[0, 1], ar[0, 1])
    n1a = list(_pass2(*exB, *hs, src2, dst2, zeros))
    x1, sx1 = _prep(e1a, n1a)
    uB, vB = gath(sx1)

    e0b = list(_pass1(*exA1, dst2, zeros))
    exB1 = _exk(uB, vB, al[1, 1], ar[1, 1])
    n0b = list(_pass2(*exA1, *x0, src2, dst2, zeros))

    e1b = list(_pass1(*exB1, dst2, zeros))
    n1b = list(_pass2(*exB1, *x1, src2, dst2, zeros))

    enc, log = _out_proj(hs, e0b, n0b, e1b, n1b, W2, b2, W3, b3)
    return (log, enc)


# speedup vs baseline: 3.8292x; 1.0039x over previous
import functools
import jax
import jax.numpy as jnp
from jax import lax
from jax.experimental import pallas as pl
from jax.experimental.pallas import tpu as pltpu
from jax.experimental.pallas import tpu_sc as plsc

_N0 = 25000
_N1 = 25000
_N = _N0 + _N1
_E = 800000
_D = 64
_G = 4
_W = _D // _G
_HEADS = 2
_CONVS = 2

_NC = 2
_NS = 16
_LM = 80
_RC = 8
_C = _LM * _RC
_ER = _E // _LM
_NCHUNK = _E // _C
_KPS = -(-_NCHUNK // _NS)
_HALF = _N // _NC
_ZR = 1568
_AROWS = _ZR * _NS
_FR = 1560


def _in_proj_body(f0_ref, f1_ref, w0_ref, b0_ref, w1_ref, b1_ref, *outs):
    i = pl.program_id(0)
    nb = pl.num_programs(0)
    half = nb // 2
    h_refs = outs[:_G]
    s_ref = outs[_G]

    def emit(h):
        for g in range(_G):
            h_refs[g][...] = h[:, g * _W:(g + 1) * _W]
        s_ref[...] = jnp.sum(h, axis=1, keepdims=True)

    @pl.when(i < half)
    def _():
        emit(jnp.dot(f0_ref[...], w0_ref[...], preferred_element_type=jnp.float32)
             + b0_ref[...])

    @pl.when(i >= half)
    def _():
        emit(jnp.dot(f1_ref[...], w1_ref[...], preferred_element_type=jnp.float32)
             + b1_ref[...])


def _in_proj(feat0, feat1, W1_0, b1_0, W1_1, b1_1):
    blk = 1000
    nb0 = _N0 // blk
    nb = 2 * nb0

    def f0_idx(i):
        return (jnp.minimum(i, nb0 - 1), 0)

    def f1_idx(i):
        return (jnp.maximum(i - nb0, 0), 0)

    outs = pl.pallas_call(
        _in_proj_body,
        grid=(nb,),
        in_specs=[
            pl.BlockSpec((blk, 128), f0_idx),
            pl.BlockSpec((blk, 128), f1_idx),
            pl.BlockSpec((128, _D), lambda i: (0, 0)),
            pl.BlockSpec((_D,), lambda i: (0,)),
            pl.BlockSpec((128, _D), lambda i: (0, 0)),
            pl.BlockSpec((_D,), lambda i: (0,)),
        ],
        out_specs=[pl.BlockSpec((blk, _W), lambda i: (i, 0)) for _ in range(_G)]
        + [pl.BlockSpec((blk, 1), lambda i: (i, 0))],
        out_shape=[jax.ShapeDtypeStruct((_N, _W), jnp.float32) for _ in range(_G)]
        + [jax.ShapeDtypeStruct((_N, 1), jnp.float32)],
    )(feat0, feat1, W1_0, b1_0, W1_1, b1_1)
    return list(outs[:_G]), outs[_G]


def _exk_body(u_ref, v_ref, al_ref, ar_ref, *ex_refs):
    x = u_ref[...] * al_ref[...] + v_ref[...] * ar_ref[...]
    x = jnp.where(x >= 0.0, x, 0.2 * x)
    ex = jnp.exp(x)
    for g in range(_G):
        ex_refs[g][...] = ex[:, g * _W:(g + 1) * _W]


def _exk(ssrc, sdst, al_c, ar_c):
    blk = 2000
    nb = _E // blk
    exs = pl.pallas_call(
        _exk_body,
        grid=(nb,),
        in_specs=[
            pl.BlockSpec((blk, 1), lambda i: (i, 0)),
            pl.BlockSpec((blk, 1), lambda i: (i, 0)),
            pl.BlockSpec((1, _D), lambda i: (0, 0)),
            pl.BlockSpec((1, _D), lambda i: (0, 0)),
        ],
        out_specs=[pl.BlockSpec((blk, _W), lambda i: (i, 0)) for _ in range(_G)],
        out_shape=[jax.ShapeDtypeStruct((_E, _W), jnp.float32) for _ in range(_G)],
    )(ssrc, sdst, al_c, ar_c)
    return list(exs)


def _prep_body(*refs):
    e_refs = refs[:_G]
    n_refs = refs[_G:2 * _G]
    x_refs = refs[2 * _G:3 * _G]
    s_ref = refs[3 * _G]
    s = None
    for g in range(_G):
        x = n_refs[g][...] / (e_refs[g][...] + 1e-9)
        x_refs[g][...] = x
        r = jnp.sum(x, axis=1, keepdims=True)
        s = r if s is None else s + r
    s_ref[...] = s


def _prep(esums, numers):
    blk = 2000
    nb = _N // blk
    dspec = pl.BlockSpec((blk, _W), lambda i: (i, 0))
    outs = pl.pallas_call(
        _prep_body,
        grid=(nb,),
        in_specs=[dspec] * (2 * _G),
        out_specs=[dspec] * _G + [pl.BlockSpec((blk, 1), lambda i: (i, 0))],
        out_shape=[jax.ShapeDtypeStruct((_N, _W), jnp.float32) for _ in range(_G)]
        + [jax.ShapeDtypeStruct((_N, 1), jnp.float32)],
    )(*esums, *numers)
    return list(outs[:_G]), outs[_G]


def _out_proj_body(*refs):
    h_refs = refs[:_G]
    e0_refs = refs[_G:2 * _G]
    n0_refs = refs[2 * _G:3 * _G]
    e1_refs = refs[3 * _G:4 * _G]
    n1_refs = refs[4 * _G:5 * _G]
    w2_ref, b2_ref, w3_ref, b3_ref, enc_ref, log_ref = refs[5 * _G:]
    cols = []
    for g in range(_G):
        cols.append(h_refs[g][...] + n0_refs[g][...] / (e0_refs[g][...] + 1e-9))
    for g in range(_G):
        cols.append(h_refs[g][...] + n1_refs[g][...] / (e1_refs[g][...] + 1e-9))
    z = jnp.concatenate(cols, axis=1)
    z = jnp.maximum(z, 0.0)
    e = jnp.dot(z, w2_ref[...], preferred_element_type=jnp.float32) + b2_ref[...]
    enc_ref[...] = e
    e = jnp.maximum(e, 0.0)
    log_ref[...] = jnp.dot(e, w3_ref[...], preferred_element_type=jnp.float32) + b3_ref[...]


def _out_proj(hs, e0, n0, e1, n1, W2, b2, W3, b3):
    blk = 1000
    nb = _N // blk
    dspec = pl.BlockSpec((blk, _W), lambda i: (i, 0))
    enc, log = pl.pallas_call(
        _out_proj_body,
        grid=(nb,),
        in_specs=[dspec] * (5 * _G) + [
            pl.BlockSpec((2 * _D, _D), lambda i: (0, 0)),
            pl.BlockSpec((_D,), lambda i: (0,)),
            pl.BlockSpec((_D, 16), lambda i: (0, 0)),
            pl.BlockSpec((16,), lambda i: (0,)),
        ],
        out_specs=[
            pl.BlockSpec((blk, _D), lambda i: (i, 0)),
            pl.BlockSpec((blk, 16), lambda i: (i, 0)),
        ],
        out_shape=[
            jax.ShapeDtypeStruct((_N, _D), jnp.float32),
            jax.ShapeDtypeStruct((_N, 16), jnp.float32),
        ],
    )(*hs, *e0, *n0, *e1, *n1, W2, b2, W3, b3)
    return enc, log


_MESH = plsc.VectorSubcoreMesh(core_axis_name="c", subcore_axis_name="s")
_SC_PARAMS = pltpu.CompilerParams(use_tc_tiling_on_sc=False)


@functools.partial(
    pl.kernel,
    out_type=[
        jax.ShapeDtypeStruct((_ER, _LM), jnp.float32),
        jax.ShapeDtypeStruct((_ER, _LM), jnp.float32),
    ],
    mesh=_MESH,
    compiler_params=_SC_PARAMS,
    scratch_types=(
        [pltpu.VMEM((_RC, _LM), jnp.int32) for _ in range(3)]
        + [pltpu.VMEM((_RC, _LM), jnp.float32) for _ in range(3)]
        + [pltpu.SemaphoreType.DMA for _ in range(6)]
    ),
)
def _gather_s(s_hbm, src2, dst2, ssrc2, sdst2, *scr):
    iv = scr[0:3]
    gv = scr[3:6]
    sem_l = scr[6:9]
    sem_g = scr[9:12]
    cid = lax.axis_index("c")
    sid = lax.axis_index("s")

    def do(idx2, out2):
        def issue_iv(c, b):
            pltpu.async_copy(idx2.at[pl.ds(c * _RC, _RC)], iv[b], sem_l[b])

        def wait_iv(b):
            pltpu.make_async_copy(idx2.at[pl.ds(0, _RC)], iv[b], sem_l[b]).wait()

        def fire_g(b):
            for j in range(_RC):
                pltpu.async_copy(s_hbm.at[iv[b].at[j]], gv[b].at[j], sem_g[b])

        def drain_g(b):
            for j in range(_RC):
                pltpu.make_async_copy(s_hbm.at[iv[b].at[j]], gv[b].at[j],
                                      sem_g[b]).wait()

        issue_iv(_idx_chunk(sid), 0)
        wait_iv(0)
        issue_iv(_idx_chunk(_NS + sid), 1)
        fire_g(0)

        def it(i, _):
            for t in range(3):
                k = i * 3 + t
                b = t
                b1 = (t + 1) % 3
                b2 = (t + 2) % 3
                c = _idx_chunk(k * _NS + sid)
                wait_iv(b1)
                issue_iv(_idx_chunk((k + 2) * _NS + sid), b2)
                fire_g(b1)
                drain_g(b)
                pltpu.sync_copy(gv[b], out2.at[pl.ds(c * _RC, _RC)])
            return 0

        lax.fori_loop(0, _KT, it, 0)
        wait_iv(1)
        drain_g(0)

    @pl.when(cid == 0)
    def _():
        do(src2, ssrc2)

    @pl.when(cid == 1)
    def _():
        do(dst2, sdst2)


def _acc_init(zeros_hbm, acc, sid):
    plsc.subcore_barrier()
    pltpu.sync_copy(zeros_hbm.at[pl.ds(sid * _ZR, _ZR)], acc.at[pl.ds(sid * _ZR, _ZR)])
    plsc.subcore_barrier()


def _acc_flush(acc, out_hbm, base_n, sid):
    plsc.subcore_barrier()
    fb = sid * _FR
    pltpu.sync_copy(acc.at[pl.ds(fb, _FR)], out_hbm.at[pl.ds(base_n + fb, _FR)])

    @pl.when(sid < (_HALF - _NS * _FR) // 8)
    def _():
        tb = _NS * _FR + sid * 8
        pltpu.sync_copy(acc.at[pl.ds(tb, 8)], out_hbm.at[pl.ds(base_n + tb, 8)])


_LM1 = 160
_C1 = _LM1 * _RC
_ER1 = _E // _LM1
_NCHUNK1 = _E // _C1
_NB = 3
_KT = 27


def _idx_chunk(c):
    return jnp.minimum(c, _NCHUNK - 1)


def _compute_idx(dstv, idxv, base_n, c, lm=_LM, nchunk=_NCHUNK):
    garb = _HALF + lax.broadcasted_iota(jnp.int32, (16,), 0)
    shift = base_n - jnp.where(c < nchunk, 0, 4 * _N)
    for j in range(_RC):
        for g in range(lm // 16):
            d = dstv[j, pl.ds(g * 16, 16)]
            loc = d - shift
            ok = (loc >= 0) & (loc < _HALF)
            idxv[j, pl.ds(g * 16, 16)] = jnp.where(ok, loc, garb)


def _fill_garbage(idxv, lm=_LM):
    garb = _HALF + lax.broadcasted_iota(jnp.int32, (16,), 0)
    for j in range(_RC):
        for g in range(lm // 16):
            idxv[j, pl.ds(g * 16, 16)] = garb


def _fire_scatter(exv, idxv, acc, sem, lm=_LM):
    for j in range(_RC):
        pltpu.async_copy(exv.at[pl.ds(j * lm, lm)], acc.at[idxv.at[j]], sem,
                         add=True)


def _drain_scatter(exv, idxv, acc, sem, lm=_LM):
    for j in range(_RC):
        pltpu.make_async_copy(exv.at[pl.ds(j * lm, lm)], acc.at[idxv.at[j]],
                              sem).wait()


def _issue_linear(ex_g, dst2, c, dstv, exv, sem, src2=None, srcv=None, cc=_C):
    rb = c * _RC
    eb = c * cc
    pltpu.async_copy(dst2.at[pl.ds(rb, _RC)], dstv, sem)
    pltpu.async_copy(ex_g.at[pl.ds(eb, cc)], exv, sem)
    if src2 is not None:
        pltpu.async_copy(src2.at[pl.ds(rb, _RC)], srcv, sem)


def _wait_linear(ex_g, dst2, dstv, exv, sem, src2=None, srcv=None, cc=_C):
    pltpu.make_async_copy(dst2.at[pl.ds(0, _RC)], dstv, sem).wait()
    pltpu.make_async_copy(ex_g.at[pl.ds(0, cc)], exv, sem).wait()
    if src2 is not None:
        pltpu.make_async_copy(src2.at[pl.ds(0, _RC)], srcv, sem).wait()


@functools.partial(
    pl.kernel,
    out_type=[jax.ShapeDtypeStruct((_N, _W), jnp.float32) for _ in range(_G)],
    mesh=_MESH,
    compiler_params=_SC_PARAMS,
    scratch_types=(
        [pltpu.VMEM((_RC, _LM1), jnp.int32) for _ in range(2 * _NB)]
        + [pltpu.VMEM((_C1, _W), jnp.float32) for _ in range(_NB)]
        + [pltpu.VMEM_SHARED((_AROWS, _W), jnp.float32)]
        + [pltpu.SemaphoreType.DMA for _ in range(2 * _NB)]
    ),
)
def _pass1(ex0, ex1, ex2, ex3, dst2w, zeros_hbm, out0, out1, out2_, out3, *scr):
    dstv = scr[0:_NB]
    idxv = scr[_NB:2 * _NB]
    exv = scr[2 * _NB:3 * _NB]
    acc = scr[3 * _NB]
    sem_l = scr[3 * _NB + 1:3 * _NB + 1 + _NB]
    sem_sc = scr[3 * _NB + 1 + _NB:3 * _NB + 1 + 2 * _NB]
    cid = lax.axis_index("c")
    sid = lax.axis_index("s")
    base_n = cid * _HALF
    exs = [ex0, ex1, ex2, ex3]
    outs = [out0, out1, out2_, out3]
    kt1 = (-(-_NCHUNK1 // _NS) + 2 + 2) // 3
    for g in range(_G):
        exg = exs[g]
        _acc_init(zeros_hbm, acc, sid)
        _fill_garbage(idxv[2], _LM1)
        _fire_scatter(exv[2], idxv[2], acc, sem_sc[2], _LM1)
        c0 = jnp.minimum(sid, _NCHUNK1 - 1)
        c1 = jnp.minimum(_NS + sid, _NCHUNK1 - 1)
        _issue_linear(exg, dst2w, c0, dstv[0], exv[0], sem_l[0], cc=_C1)
        _wait_linear(exg, dst2w, dstv[0], exv[0], sem_l[0], cc=_C1)
        _issue_linear(exg, dst2w, c1, dstv[1], exv[1], sem_l[1], cc=_C1)

        def it(i, _, exg=exg):
            for t in range(3):
                k = i * 3 + t
                b = t
                b1 = (t + 1) % 3
                b2 = (t + 2) % 3
                c = k * _NS + sid
                c2 = jnp.minimum((k + 2) * _NS + sid, _NCHUNK1 - 1)
                _drain_scatter(exv[b2], idxv[b2], acc, sem_sc[b2], _LM1)
                _wait_linear(exg, dst2w, dstv[b1], exv[b1], sem_l[b1], cc=_C1)
                _issue_linear(exg, dst2w, c2, dstv[b2], exv[b2], sem_l[b2],
                              cc=_C1)
                _compute_idx(dstv[b], idxv[b], base_n, c, _LM1, _NCHUNK1)
                _fire_scatter(exv[b], idxv[b], acc, sem_sc[b], _LM1)
            return 0

        lax.fori_loop(0, kt1, it, 0)
        _drain_scatter(exv[2], idxv[2], acc, sem_sc[2], _LM1)
        _wait_linear(exg, dst2w, dstv[1], exv[1], sem_l[1], cc=_C1)
        _acc_flush(acc, outs[g], base_n, sid)


@functools.partial(
    pl.kernel,
    out_type=[jax.ShapeDtypeStruct((_N, _W), jnp.float32) for _ in range(_G)],
    mesh=_MESH,
    compiler_params=_SC_PARAMS,
    scratch_types=(
        [pltpu.VMEM((_RC, _LM), jnp.int32) for _ in range(3 * _NB)]
        + [pltpu.VMEM((_C, _W), jnp.float32) for _ in range(2 * _NB)]
        + [pltpu.VMEM_SHARED((_AROWS, _W), jnp.float32)]
        + [pltpu.SemaphoreType.DMA for _ in range(3 * _NB)]
    ),
)
def _pass2(ex0, ex1, ex2, ex3, h0, h1, h2, h3, src2, dst2, zeros_hbm,
           out0, out1, out2_, out3, *scr):
    srcv = scr[0:_NB]
    dstv = scr[_NB:2 * _NB]
    idxv = scr[2 * _NB:3 * _NB]
    exv = scr[3 * _NB:4 * _NB]
    hv = scr[4 * _NB:5 * _NB]
    acc = scr[5 * _NB]
    sem_l = scr[5 * _NB + 1:5 * _NB + 1 + _NB]
    sem_sc = scr[5 * _NB + 1 + _NB:5 * _NB + 1 + 2 * _NB]
    sem_h = scr[5 * _NB + 1 + 2 * _NB:5 * _NB + 1 + 3 * _NB]
    cid = lax.axis_index("c")
    sid = lax.axis_index("s")
    base_n = cid * _HALF
    exs = [ex0, ex1, ex2, ex3]
    hs = [h0, h1, h2, h3]
    outs = [out0, out1, out2_, out3]

    def fire_h(hg, sv, hb, sem):
        for j in range(_RC):
            pltpu.async_copy(hg.at[sv.at[j]], hb.at[pl.ds(j * _LM, _LM)], sem)

    def drain_h(hg, sv, hb, sem):
        for j in range(_RC):
            pltpu.make_async_copy(hg.at[sv.at[j]], hb.at[pl.ds(j * _LM, _LM)],
                                  sem).wait()

    for g in range(_G):
        exg = exs[g]
        hg = hs[g]
        _acc_init(zeros_hbm, acc, sid)
        _fill_garbage(idxv[2])
        _fire_scatter(exv[2], idxv[2], acc, sem_sc[2])
        _issue_linear(exg, dst2, _idx_chunk(sid), dstv[0], exv[0], sem_l[0],
                      src2, srcv[0])
        _wait_linear(exg, dst2, dstv[0], exv[0], sem_l[0], src2, srcv[0])
        _issue_linear(exg, dst2, _idx_chunk(16 + sid), dstv[1], exv[1],
                      sem_l[1], src2, srcv[1])
        fire_h(hg, srcv[0], hv[0], sem_h[0])

        def it(i, _, exg=exg, hg=hg):
            for t in range(3):
                k = i * 3 + t
                b = t
                b1 = (t + 1) % 3
                b2 = (t + 2) % 3
                c = k * _NS + sid
                _drain_scatter(exv[b2], idxv[b2], acc, sem_sc[b2])
                _wait_linear(exg, dst2, dstv[b1], exv[b1], sem_l[b1],
                             src2, srcv[b1])
                _issue_linear(exg, dst2, _idx_chunk((k + 2) * _NS + sid),
                              dstv[b2], exv[b2], sem_l[b2], src2, srcv[b2])
                fire_h(hg, srcv[b1], hv[b1], sem_h[b1])
                drain_h(hg, srcv[b], hv[b], sem_h[b])
                _compute_idx(dstv[b], idxv[b], base_n, c)

                def mul_row(r8, _):
                    for u in range(8):
                        r = r8 * 8 + u
                        exv[b][r, pl.ds(0, 16)] = (exv[b][r, pl.ds(0, 16)]
                                                   * hv[b][r, pl.ds(0, 16)])
                    return 0

                lax.fori_loop(0, _C // 8, mul_row, 0)
                _fire_scatter(exv[b], idxv[b], acc, sem_sc[b])
            return 0

        lax.fori_loop(0, _KT, it, 0)
        _drain_scatter(exv[2], idxv[2], acc, sem_sc[2])
        _wait_linear(exg, dst2, dstv[1], exv[1], sem_l[1], src2, srcv[1])
        drain_h(hg, srcv[0], hv[0], sem_h[0])
        _acc_flush(acc, outs[g], base_n, sid)


def kernel(feat0, feat1, edge_index, e_feat, W1_0, b1_0, W1_1, b1_1, al, ar, W2, b2, W3, b3):
    src2 = edge_index[0].reshape(_ER, _LM)
    dst2 = edge_index[1].reshape(_ER, _LM)
    dst2w = edge_index[1].reshape(_ER1, _LM1)
    zeros = jnp.zeros((_AROWS, _W), jnp.float32)
    hs, s0 = _in_proj(feat0, feat1, W1_0, b1_0, W1_1, b1_1)

    def gath(s_n1):
        u2, v2 = _gather_s(s_n1.reshape(_N), src2, dst2)
        return u2.reshape(_E, 1), v2.reshape(_E, 1)

    u0, v0 = gath(s0)
    exA = _exk(u0, v0, al[0, 0], ar[0, 0])
    exB = _exk(u0, v0, al[1, 0], ar[1, 0])

    e0a = list(_pass1(*exA, dst2w, zeros))
    n0a = list(_pass2(*exA, *hs, src2, dst2, zeros))
    x0, sx0 = _prep(e0a, n0a)
    uA, vA = gath(sx0)

    e1a = list(_pass1(*exB, dst2w, zeros))
    exA1 = _exk(uA, vA, al[0, 1], ar[0, 1])
    n1a = list(_pass2(*exB, *hs, src2, dst2, zeros))
    x1, sx1 = _prep(e1a, n1a)
    uB, vB = gath(sx1)

    e0b = list(_pass1(*exA1, dst2w, zeros))
    exB1 = _exk(uB, vB, al[1, 1], ar[1, 1])
    n0b = list(_pass2(*exA1, *x0, src2, dst2, zeros))

    e1b = list(_pass1(*exB1, dst2w, zeros))
    n1b = list(_pass2(*exB1, *x1, src2, dst2, zeros))

    enc, log = _out_proj(hs, e0b, n0b, e1b, n1b, W2, b2, W3, b3)
    return (log, enc)
